# baseline (speedup 1.0000x reference)
import jax
import jax.numpy as jnp
from jax import lax
from jax.experimental import pallas as pl
from jax.experimental.pallas import tpu as pltpu
from jax.experimental.pallas import tpu_sc as plsc

N = 10000
E = 320000
C = 128
NTILE = 16
NPAD = 10240
NSLC = NPAD // NTILE
EPT = E // NTILE
B = 128
NFULL = EPT // B
TAIL = EPT - NFULL * B
CG = 4


def _sc_body(feat2, src_hbm, dst_hbm, eps_r, out_r,
             eps_v, rows_v, msg_v, src_v, dst_v, ones_m, degc_v,
             epst_v, rowst_v, msgt_v, srct_v, dstt_v,
             s_sh, t_sh, deg_sh,
             sem_a, sem_b, sem_c, sem_g, sem_x, sem_y, sem_z):
  core = lax.axis_index("c")
  sub = lax.axis_index("s")
  ch0 = core * 64
  n0 = sub * NSLC
  e_base = sub * EPT
  offv = jnp.full((16,), core * NPAD, jnp.int32)

  @plsc.parallel_loop(0, B)
  def _z(r):
    for j in range(CG):
      msg_v[r, pl.ds(j * 16, 16)] = jnp.zeros((16,), jnp.float32)
    ones_m[r] = jnp.ones((16,), jnp.float32)
    degc_v[r] = jnp.zeros((16,), jnp.float32)

  def _init(q, _):
    off = n0 + q * B
    pltpu.sync_copy(msg_v, s_sh.at[pl.ds(off, B)])
    pltpu.sync_copy(msg_v, t_sh.at[pl.ds(off, B)])
    pltpu.sync_copy(degc_v, deg_sh.at[pl.ds(off, B)])
    return 0
  lax.fori_loop(0, NSLC // B, _init, 0)

  plsc.subcore_barrier()

  def _chunk(k, _):
    e0 = e_base + k * B
    la = pltpu.async_copy(src_hbm.at[pl.ds(e0, B)], src_v, sem_a)
    lb = pltpu.async_copy(dst_hbm.at[pl.ds(e0, B)], dst_v, sem_b)
    lc = pltpu.async_copy(
        eps_r.at[pl.ds(e0, B), pl.ds(ch0, 64)], eps_v, sem_c)
    la.wait()

    @plsc.parallel_loop(0, B // 16)
    def _o(i):
      src_v[pl.ds(i * 16, 16)] = src_v[pl.ds(i * 16, 16)] + offv

    g = pltpu.async_copy(feat2.at[src_v], rows_v, sem_g)
    lb.wait()
    lc.wait()
    g.wait()

    @plsc.parallel_loop(0, B, unroll=4)
    def _compute(r):
      for j in range(CG):
        w = jnp.maximum(eps_v[r, pl.ds(j * 16, 16)] + 1.0, 0.0)
        eps_v[r, pl.ds(j * 16, 16)] = w
        msg_v[r, pl.ds(j * 16, 16)] = w * rows_v[r, pl.ds(j * 16, 16)]

    sx = pltpu.async_copy(eps_v, s_sh.at[dst_v], sem_x, add=True)
    sy = pltpu.async_copy(msg_v, t_sh.at[dst_v], sem_y, add=True)
    sz = pltpu.async_copy(ones_m, deg_sh.at[dst_v], sem_z, add=True)
    sx.wait()
    sy.wait()
    sz.wait()
    return 0
  lax.fori_loop(0, NFULL, _chunk, 0)

  e0 = e_base + NFULL * B
  pltpu.sync_copy(src_hbm.at[pl.ds(e0, TAIL)], srct_v)
  pltpu.sync_copy(dst_hbm.at[pl.ds(e0, TAIL)], dstt_v)
  pltpu.sync_copy(eps_r.at[pl.ds(e0, TAIL), pl.ds(ch0, 64)], epst_v)

  @plsc.parallel_loop(0, TAIL // 16)
  def _ot(i):
    srct_v[pl.ds(i * 16, 16)] = srct_v[pl.ds(i * 16, 16)] + offv

  pltpu.async_copy(feat2.at[srct_v], rowst_v, sem_g).wait()

  @plsc.parallel_loop(0, TAIL)
  def _compute_t(r):
    for j in range(CG):
      w = jnp.maximum(epst_v[r, pl.ds(j * 16, 16)] + 1.0, 0.0)
      epst_v[r, pl.ds(j * 16, 16)] = w
      msgt_v[r, pl.ds(j * 16, 16)] = w * rowst_v[r, pl.ds(j * 16, 16)]

  pltpu.sync_copy(epst_v, s_sh.at[dstt_v], add=True)
  pltpu.sync_copy(msgt_v, t_sh.at[dstt_v], add=True)
  pltpu.sync_copy(ones_m.at[pl.ds(0, TAIL)], deg_sh.at[dstt_v], add=True)

  plsc.subcore_barrier()

  def _norm(q, _):
    off = n0 + q * B
    pltpu.sync_copy(s_sh.at[pl.ds(off, B)], eps_v)
    pltpu.sync_copy(t_sh.at[pl.ds(off, B)], rows_v)
    pltpu.sync_copy(deg_sh.at[pl.ds(off, B)], degc_v)

    @plsc.parallel_loop(0, B)
    def _rows(r):
      d = degc_v[r]
      for j in range(CG):
        s = eps_v[r, pl.ds(j * 16, 16)]
        nz = s != 0.0
        sc = jnp.where(nz, d / jnp.where(nz, s, 1.0), 0.0)
        msg_v[r, pl.ds(j * 16, 16)] = sc * rows_v[r, pl.ds(j * 16, 16)]

    pltpu.sync_copy(msg_v, out_r.at[pl.ds(off, B), pl.ds(ch0, 64)])
    return 0
  lax.fori_loop(0, NSLC // B, _norm, 0)


_sc_kernel = pl.kernel(
    _sc_body,
    out_type=jax.ShapeDtypeStruct((NPAD, 128), jnp.float32),
    mesh=plsc.VectorSubcoreMesh(core_axis_name="c", subcore_axis_name="s"),
    compiler_params=pltpu.CompilerParams(use_tc_tiling_on_sc=False),
    scratch_types=[
        pltpu.VMEM((B, 64), jnp.float32),
        pltpu.VMEM((B, 64), jnp.float32),
        pltpu.VMEM((B, 64), jnp.float32),
        pltpu.VMEM((B,), jnp.int32),
        pltpu.VMEM((B,), jnp.int32),
        pltpu.VMEM((B, 16), jnp.float32),
        pltpu.VMEM((B, 16), jnp.float32),
        pltpu.VMEM((TAIL, 64), jnp.float32),
        pltpu.VMEM((TAIL, 64), jnp.float32),
        pltpu.VMEM((TAIL, 64), jnp.float32),
        pltpu.VMEM((TAIL,), jnp.int32),
        pltpu.VMEM((TAIL,), jnp.int32),
        pltpu.VMEM_SHARED((NPAD, 64), jnp.float32),
        pltpu.VMEM_SHARED((NPAD, 64), jnp.float32),
        pltpu.VMEM_SHARED((NPAD, 16), jnp.float32),
        pltpu.SemaphoreType.DMA,
        pltpu.SemaphoreType.DMA,
        pltpu.SemaphoreType.DMA,
        pltpu.SemaphoreType.DMA,
        pltpu.SemaphoreType.DMA,
        pltpu.SemaphoreType.DMA,
        pltpu.SemaphoreType.DMA,
    ],
)


def _mm_body(a_ref, w_ref, b_ref, o_ref):
  o_ref[...] = (
      jnp.dot(a_ref[...], w_ref[...], preferred_element_type=jnp.float32)
      + b_ref[0:1, :]
  )


def _matmul(agg_pad, W, b2):
  return pl.pallas_call(
      _mm_body,
      grid=(10,),
      in_specs=[
          pl.BlockSpec((1000, C), lambda i: (i, 0)),
          pl.BlockSpec((C, C), lambda i: (0, 0)),
          pl.BlockSpec((8, C), lambda i: (0, 0)),
      ],
      out_specs=pl.BlockSpec((1000, C), lambda i: (i, 0)),
      out_shape=jax.ShapeDtypeStruct((N, C), jnp.float32),
  )(agg_pad, W, b2)


def kernel(feat, edge_index, eps, W, b):
  feat_pad = jnp.concatenate(
      [feat, jnp.zeros((NPAD - N, C), feat.dtype)], axis=0)
  feat2 = jnp.concatenate(
      [feat_pad[:, :64], feat_pad[:, 64:]], axis=0)
  eps_r = eps
  agg = _sc_kernel(
      feat2, edge_index[0], edge_index[1], eps_r).reshape(NPAD, C)
  b2 = jnp.broadcast_to(b.reshape(1, C), (8, C))
  return _matmul(agg, W, b2)


# speedup vs baseline: 1.1075x; 1.1075x over previous
import jax
import jax.numpy as jnp
from jax import lax
from jax.experimental import pallas as pl
from jax.experimental.pallas import tpu as pltpu
from jax.experimental.pallas import tpu_sc as plsc

N = 10000
E = 320000
C = 128
NTILE = 16
NPAD = 10240
NSLC = NPAD // NTILE
EPT = E // NTILE
B = 128
NFULL = EPT // B
TAIL = EPT - NFULL * B
CG = 4


def _sc_body(feat2, src_hbm, dst_hbm, eps_r, out_r,
             eps_v, rows_v, msg_v, src_v, dst_v, ones_m, degc_v,
             epst_v, rowst_v, msgt_v, srct_v, dstt_v,
             s_sh, t_sh, deg_sh,
             sem_a, sem_b, sem_c, sem_g, sem_x, sem_y, sem_z):
  core = lax.axis_index("c")
  sub = lax.axis_index("s")
  ch0 = core * 64
  n0 = sub * NSLC
  e_base = sub * EPT
  offv = jnp.full((16,), core * NPAD, jnp.int32)

  @plsc.parallel_loop(0, B)
  def _z(r):
    for j in range(CG):
      msg_v[r, pl.ds(j * 16, 16)] = jnp.zeros((16,), jnp.float32)
    ones_m[r] = jnp.ones((16,), jnp.float32)
    degc_v[r] = jnp.zeros((16,), jnp.float32)

  def _init(q, _):
    off = n0 + q * B
    pltpu.sync_copy(msg_v, s_sh.at[pl.ds(off, B)])
    pltpu.sync_copy(msg_v, t_sh.at[pl.ds(off, B)])
    pltpu.sync_copy(degc_v, deg_sh.at[pl.ds(off, B)])
    return 0
  lax.fori_loop(0, NSLC // B, _init, 0)

  plsc.subcore_barrier()

  def _chunk(k, _):
    e0 = e_base + k * B
    la = pltpu.async_copy(src_hbm.at[pl.ds(e0, B)], src_v, sem_a)
    lb = pltpu.async_copy(dst_hbm.at[pl.ds(e0, B)], dst_v, sem_b)
    lc = pltpu.async_copy(
        eps_r.at[pl.ds(e0, B), pl.ds(ch0, 64)], eps_v, sem_c)
    la.wait()

    @plsc.parallel_loop(0, B // 16)
    def _o(i):
      src_v[pl.ds(i * 16, 16)] = src_v[pl.ds(i * 16, 16)] + offv

    g = pltpu.async_copy(feat2.at[src_v], rows_v, sem_g)
    lc.wait()

    @plsc.parallel_loop(0, B, unroll=4)
    def _compute_w(r):
      for j in range(CG):
        eps_v[r, pl.ds(j * 16, 16)] = jnp.maximum(
            eps_v[r, pl.ds(j * 16, 16)] + 1.0, 0.0)

    lb.wait()
    sx = pltpu.async_copy(eps_v, s_sh.at[dst_v], sem_x, add=True)
    sz = pltpu.async_copy(ones_m, deg_sh.at[dst_v], sem_z, add=True)
    g.wait()

    @plsc.parallel_loop(0, B, unroll=4)
    def _compute_m(r):
      for j in range(CG):
        msg_v[r, pl.ds(j * 16, 16)] = (
            eps_v[r, pl.ds(j * 16, 16)] * rows_v[r, pl.ds(j * 16, 16)])

    sy = pltpu.async_copy(msg_v, t_sh.at[dst_v], sem_y, add=True)
    sx.wait()
    sz.wait()
    sy.wait()
    return 0
  lax.fori_loop(0, NFULL, _chunk, 0)

  e0 = e_base + NFULL * B
  pltpu.sync_copy(src_hbm.at[pl.ds(e0, TAIL)], srct_v)
  pltpu.sync_copy(dst_hbm.at[pl.ds(e0, TAIL)], dstt_v)
  pltpu.sync_copy(eps_r.at[pl.ds(e0, TAIL), pl.ds(ch0, 64)], epst_v)

  @plsc.parallel_loop(0, TAIL // 16)
  def _ot(i):
    srct_v[pl.ds(i * 16, 16)] = srct_v[pl.ds(i * 16, 16)] + offv

  pltpu.async_copy(feat2.at[srct_v], rowst_v, sem_g).wait()

  @plsc.parallel_loop(0, TAIL)
  def _compute_t(r):
    for j in range(CG):
      w = jnp.maximum(epst_v[r, pl.ds(j * 16, 16)] + 1.0, 0.0)
      epst_v[r, pl.ds(j * 16, 16)] = w
      msgt_v[r, pl.ds(j * 16, 16)] = w * rowst_v[r, pl.ds(j * 16, 16)]

  pltpu.sync_copy(epst_v, s_sh.at[dstt_v], add=True)
  pltpu.sync_copy(msgt_v, t_sh.at[dstt_v], add=True)
  pltpu.sync_copy(ones_m.at[pl.ds(0, TAIL)], deg_sh.at[dstt_v], add=True)

  plsc.subcore_barrier()

  def _norm(q, _):
    off = n0 + q * B
    pltpu.sync_copy(s_sh.at[pl.ds(off, B)], eps_v)
    pltpu.sync_copy(t_sh.at[pl.ds(off, B)], rows_v)
    pltpu.sync_copy(deg_sh.at[pl.ds(off, B)], degc_v)

    @plsc.parallel_loop(0, B)
    def _rows(r):
      d = degc_v[r]
      for j in range(CG):
        s = eps_v[r, pl.ds(j * 16, 16)]
        nz = s != 0.0
        sc = jnp.where(nz, d / jnp.where(nz, s, 1.0), 0.0)
        msg_v[r, pl.ds(j * 16, 16)] = sc * rows_v[r, pl.ds(j * 16, 16)]

    pltpu.sync_copy(msg_v, out_r.at[pl.ds(off, B), pl.ds(ch0, 64)])
    return 0
  lax.fori_loop(0, NSLC // B, _norm, 0)


_sc_kernel = pl.kernel(
    _sc_body,
    out_type=jax.ShapeDtypeStruct((NPAD, 128), jnp.float32),
    mesh=plsc.VectorSubcoreMesh(core_axis_name="c", subcore_axis_name="s"),
    compiler_params=pltpu.CompilerParams(use_tc_tiling_on_sc=False),
    scratch_types=[
        pltpu.VMEM((B, 64), jnp.float32),
        pltpu.VMEM((B, 64), jnp.float32),
        pltpu.VMEM((B, 64), jnp.float32),
        pltpu.VMEM((B,), jnp.int32),
        pltpu.VMEM((B,), jnp.int32),
        pltpu.VMEM((B, 16), jnp.float32),
        pltpu.VMEM((B, 16), jnp.float32),
        pltpu.VMEM((TAIL, 64), jnp.float32),
        pltpu.VMEM((TAIL, 64), jnp.float32),
        pltpu.VMEM((TAIL, 64), jnp.float32),
        pltpu.VMEM((TAIL,), jnp.int32),
        pltpu.VMEM((TAIL,), jnp.int32),
        pltpu.VMEM_SHARED((NPAD, 64), jnp.float32),
        pltpu.VMEM_SHARED((NPAD, 64), jnp.float32),
        pltpu.VMEM_SHARED((NPAD, 16), jnp.float32),
        pltpu.SemaphoreType.DMA,
        pltpu.SemaphoreType.DMA,
        pltpu.SemaphoreType.DMA,
        pltpu.SemaphoreType.DMA,
        pltpu.SemaphoreType.DMA,
        pltpu.SemaphoreType.DMA,
        pltpu.SemaphoreType.DMA,
    ],
)


def _mm_body(a_ref, w_ref, b_ref, o_ref):
  o_ref[...] = (
      jnp.dot(a_ref[...], w_ref[...], preferred_element_type=jnp.float32)
      + b_ref[0:1, :]
  )


def _matmul(agg_pad, W, b2):
  return pl.pallas_call(
      _mm_body,
      grid=(10,),
      in_specs=[
          pl.BlockSpec((1000, C), lambda i: (i, 0)),
          pl.BlockSpec((C, C), lambda i: (0, 0)),
          pl.BlockSpec((8, C), lambda i: (0, 0)),
      ],
      out_specs=pl.BlockSpec((1000, C), lambda i: (i, 0)),
      out_shape=jax.ShapeDtypeStruct((N, C), jnp.float32),
  )(agg_pad, W, b2)


def kernel(feat, edge_index, eps, W, b):
  feat_pad = jnp.concatenate(
      [feat, jnp.zeros((NPAD - N, C), feat.dtype)], axis=0)
  feat2 = jnp.concatenate(
      [feat_pad[:, :64], feat_pad[:, 64:]], axis=0)
  eps_r = eps
  agg = _sc_kernel(
      feat2, edge_index[0], edge_index[1], eps_r).reshape(NPAD, C)
  b2 = jnp.broadcast_to(b.reshape(1, C), (8, C))
  return _matmul(agg, W, b2)


# speedup vs baseline: 1.1163x; 1.0079x over previous
import jax
import jax.numpy as jnp
from jax import lax
from jax.experimental import pallas as pl
from jax.experimental.pallas import tpu as pltpu
from jax.experimental.pallas import tpu_sc as plsc

N = 10000
E = 320000
C = 128
NTILE = 16
NPAD = 10240
NSLC = NPAD // NTILE
EPT = E // NTILE
B = 128
NFULL = EPT // B
TAIL = EPT - NFULL * B
CG = 4


def _sc_body(feat2, src_hbm, dst_hbm, eps_r, out_r,
             eps_v, rows_v, msg_v, src_v, dst_v, ones_v, zer_v,
             epst_v, rowst_v, msgt_v, srct_v, dstt_v,
             deg_sm, s_sh, t_sh, deg_sh,
             sem_a, sem_b, sem_c, sem_g, sem_x, sem_y, sem_z):
  core = lax.axis_index("c")
  sub = lax.axis_index("s")
  ch0 = core * 64
  n0 = sub * NSLC
  e_base = sub * EPT
  offv = jnp.full((16,), core * NPAD, jnp.int32)

  @plsc.parallel_loop(0, B)
  def _z(r):
    for j in range(CG):
      msg_v[r, pl.ds(j * 16, 16)] = jnp.zeros((16,), jnp.float32)

  @plsc.parallel_loop(0, 40)
  def _z1(i):
    zer_v[pl.ds(i * 16, 16)] = jnp.zeros((16,), jnp.float32)

  @plsc.parallel_loop(0, B // 16)
  def _z2(i):
    ones_v[pl.ds(i * 16, 16)] = jnp.ones((16,), jnp.float32)

  def _init(q, _):
    off = n0 + q * B
    pltpu.sync_copy(msg_v, s_sh.at[pl.ds(off, B)])
    pltpu.sync_copy(msg_v, t_sh.at[pl.ds(off, B)])
    return 0
  lax.fori_loop(0, NSLC // B, _init, 0)
  pltpu.sync_copy(zer_v.at[pl.ds(0, NSLC)], deg_sh.at[pl.ds(n0, NSLC)])

  plsc.subcore_barrier()

  def _issue_eps(kk):
    e0 = jnp.where(kk < NFULL, e_base + kk * B, e_base)
    base = (kk % 2) * B
    pltpu.async_copy(
        eps_r.at[pl.ds(e0, B), pl.ds(ch0, 64)], eps_v.at[pl.ds(base, B)],
        sem_c)

  def _drain_eps():
    pltpu.make_async_copy(
        eps_r.at[pl.ds(e_base, B), pl.ds(ch0, 64)],
        eps_v.at[pl.ds(0, B)], sem_c).wait()

  _issue_eps(0)

  def _chunk(k, _):
    e0 = e_base + k * B
    base = (k % 2) * B
    la = pltpu.async_copy(src_hbm.at[pl.ds(e0, B)], src_v, sem_a)
    lb = pltpu.async_copy(dst_hbm.at[pl.ds(e0, B)], dst_v, sem_b)
    la.wait()

    @plsc.parallel_loop(0, B // 16)
    def _o(i):
      src_v[pl.ds(i * 16, 16)] = src_v[pl.ds(i * 16, 16)] + offv

    g = pltpu.async_copy(feat2.at[src_v], rows_v, sem_g)
    _drain_eps()

    @plsc.parallel_loop(0, B, unroll=4)
    def _compute_w(r):
      for j in range(CG):
        eps_v[base + r, pl.ds(j * 16, 16)] = jnp.maximum(
            eps_v[base + r, pl.ds(j * 16, 16)] + 1.0, 0.0)

    lb.wait()
    sx = pltpu.async_copy(
        eps_v.at[pl.ds(base, B)], s_sh.at[dst_v], sem_x, add=True)
    sz = pltpu.async_copy(ones_v, deg_sh.at[dst_v], sem_z, add=True)
    _issue_eps(k + 1)
    g.wait()

    @plsc.parallel_loop(0, B, unroll=4)
    def _compute_m(r):
      for j in range(CG):
        msg_v[r, pl.ds(j * 16, 16)] = (
            eps_v[base + r, pl.ds(j * 16, 16)] * rows_v[r, pl.ds(j * 16, 16)])

    sy = pltpu.async_copy(msg_v, t_sh.at[dst_v], sem_y, add=True)
    sx.wait()
    sz.wait()
    sy.wait()
    return 0
  lax.fori_loop(0, NFULL, _chunk, 0)
  _drain_eps()

  e0 = e_base + NFULL * B
  pltpu.sync_copy(src_hbm.at[pl.ds(e0, TAIL)], srct_v)
  pltpu.sync_copy(dst_hbm.at[pl.ds(e0, TAIL)], dstt_v)
  pltpu.sync_copy(eps_r.at[pl.ds(e0, TAIL), pl.ds(ch0, 64)], epst_v)

  @plsc.parallel_loop(0, TAIL // 16)
  def _ot(i):
    srct_v[pl.ds(i * 16, 16)] = srct_v[pl.ds(i * 16, 16)] + offv

  pltpu.async_copy(feat2.at[srct_v], rowst_v, sem_g).wait()

  @plsc.parallel_loop(0, TAIL)
  def _compute_t(r):
    for j in range(CG):
      w = jnp.maximum(epst_v[r, pl.ds(j * 16, 16)] + 1.0, 0.0)
      epst_v[r, pl.ds(j * 16, 16)] = w
      msgt_v[r, pl.ds(j * 16, 16)] = w * rowst_v[r, pl.ds(j * 16, 16)]

  pltpu.sync_copy(epst_v, s_sh.at[dstt_v], add=True)
  pltpu.sync_copy(msgt_v, t_sh.at[dstt_v], add=True)
  pltpu.sync_copy(ones_v.at[pl.ds(0, TAIL)], deg_sh.at[dstt_v], add=True)

  plsc.subcore_barrier()

  pltpu.sync_copy(deg_sh.at[pl.ds(n0, NSLC)], deg_sm)

  def _norm(q, _):
    off = n0 + q * B
    pltpu.sync_copy(s_sh.at[pl.ds(off, B)], eps_v.at[pl.ds(0, B)])
    pltpu.sync_copy(t_sh.at[pl.ds(off, B)], rows_v)

    @plsc.parallel_loop(0, B)
    def _rows(r):
      d = jnp.full((16,), deg_sm[q * B + r], jnp.float32)
      for j in range(CG):
        s = eps_v[r, pl.ds(j * 16, 16)]
        nz = s != 0.0
        sc = jnp.where(nz, d / jnp.where(nz, s, 1.0), 0.0)
        msg_v[r, pl.ds(j * 16, 16)] = sc * rows_v[r, pl.ds(j * 16, 16)]

    pltpu.sync_copy(msg_v, out_r.at[pl.ds(off, B), pl.ds(ch0, 64)])
    return 0
  lax.fori_loop(0, NSLC // B, _norm, 0)


_sc_kernel = pl.kernel(
    _sc_body,
    out_type=jax.ShapeDtypeStruct((NPAD, 128), jnp.float32),
    mesh=plsc.VectorSubcoreMesh(core_axis_name="c", subcore_axis_name="s"),
    compiler_params=pltpu.CompilerParams(use_tc_tiling_on_sc=False),
    scratch_types=[
        pltpu.VMEM((2 * B, 64), jnp.float32),
        pltpu.VMEM((B, 64), jnp.float32),
        pltpu.VMEM((B, 64), jnp.float32),
        pltpu.VMEM((B,), jnp.int32),
        pltpu.VMEM((B,), jnp.int32),
        pltpu.VMEM((B,), jnp.float32),
        pltpu.VMEM((640,), jnp.float32),
        pltpu.VMEM((TAIL, 64), jnp.float32),
        pltpu.VMEM((TAIL, 64), jnp.float32),
        pltpu.VMEM((TAIL, 64), jnp.float32),
        pltpu.VMEM((TAIL,), jnp.int32),
        pltpu.VMEM((TAIL,), jnp.int32),
        pltpu.SMEM((NSLC,), jnp.float32),
        pltpu.VMEM_SHARED((NPAD, 64), jnp.float32),
        pltpu.VMEM_SHARED((NPAD, 64), jnp.float32),
        pltpu.VMEM_SHARED((NPAD,), jnp.float32),
        pltpu.SemaphoreType.DMA,
        pltpu.SemaphoreType.DMA,
        pltpu.SemaphoreType.DMA,
        pltpu.SemaphoreType.DMA,
        pltpu.SemaphoreType.DMA,
        pltpu.SemaphoreType.DMA,
        pltpu.SemaphoreType.DMA,
    ],
)


def _mm_body(a_ref, w_ref, b_ref, o_ref):
  o_ref[...] = (
      jnp.dot(a_ref[...], w_ref[...], preferred_element_type=jnp.float32)
      + b_ref[0:1, :]
  )


def _matmul(agg_pad, W, b2):
  return pl.pallas_call(
      _mm_body,
      grid=(10,),
      in_specs=[
          pl.BlockSpec((1000, C), lambda i: (i, 0)),
          pl.BlockSpec((C, C), lambda i: (0, 0)),
          pl.BlockSpec((8, C), lambda i: (0, 0)),
      ],
      out_specs=pl.BlockSpec((1000, C), lambda i: (i, 0)),
      out_shape=jax.ShapeDtypeStruct((N, C), jnp.float32),
  )(agg_pad, W, b2)


def kernel(feat, edge_index, eps, W, b):
  feat_pad = jnp.concatenate(
      [feat, jnp.zeros((NPAD - N, C), feat.dtype)], axis=0)
  feat2 = jnp.concatenate(
      [feat_pad[:, :64], feat_pad[:, 64:]], axis=0)
  eps_r = eps
  agg = _sc_kernel(
      feat2, edge_index[0], edge_index[1], eps_r).reshape(NPAD, C)
  b2 = jnp.broadcast_to(b.reshape(1, C), (8, C))
  return _matmul(agg, W, b2)


# speedup vs baseline: 1.1613x; 1.0404x over previous
import jax
import jax.numpy as jnp
from jax import lax
from jax.experimental import pallas as pl
from jax.experimental.pallas import tpu as pltpu
from jax.experimental.pallas import tpu_sc as plsc

N = 10000
E = 320000
C = 128
NTILE = 16
NPAD = 10240
NSLC = NPAD // NTILE
EPT = E // NTILE
B = 128
NFULL = EPT // B
TAIL = EPT - NFULL * B
CG = 4


def _sc_body(feat2, src_hbm, dst_hbm, eps_r, out_r,
             eps_v, rows_v, msg_v, src_v, dst_v, ones_v, zer_v,
             epst_v, rowst_v, msgt_v, srct_v, dstt_v,
             deg_sm, s_sh, t_sh, deg_sh,
             sem_a, sem_b, sem_c, sem_g, sem_h, sem_x, sem_y, sem_z):
  core = lax.axis_index("c")
  sub = lax.axis_index("s")
  ch0 = core * 64
  n0 = sub * NSLC
  e_base = sub * EPT
  offv = jnp.full((16,), core * NPAD, jnp.int32)

  @plsc.parallel_loop(0, B)
  def _z(r):
    for j in range(CG):
      msg_v[r, pl.ds(j * 16, 16)] = jnp.zeros((16,), jnp.float32)

  @plsc.parallel_loop(0, 40)
  def _z1(i):
    zer_v[pl.ds(i * 16, 16)] = jnp.zeros((16,), jnp.float32)

  @plsc.parallel_loop(0, B // 16)
  def _z2(i):
    ones_v[pl.ds(i * 16, 16)] = jnp.ones((16,), jnp.float32)

  def _init(q, _):
    off = n0 + q * B
    pltpu.sync_copy(msg_v, s_sh.at[pl.ds(off, B)])
    pltpu.sync_copy(msg_v, t_sh.at[pl.ds(off, B)])
    return 0
  lax.fori_loop(0, NSLC // B, _init, 0)
  pltpu.sync_copy(zer_v.at[pl.ds(0, NSLC)], deg_sh.at[pl.ds(n0, NSLC)])

  plsc.subcore_barrier()

  def _issue_eps(kk):
    e0 = jnp.where(kk < NFULL, e_base + kk * B, e_base)
    base = (kk % 2) * B
    pltpu.async_copy(
        eps_r.at[pl.ds(e0, B), pl.ds(ch0, 64)], eps_v.at[pl.ds(base, B)],
        sem_c)

  def _drain_eps():
    pltpu.make_async_copy(
        eps_r.at[pl.ds(e_base, B), pl.ds(ch0, 64)],
        eps_v.at[pl.ds(0, B)], sem_c).wait()

  _issue_eps(0)

  def _chunk(k, _):
    e0 = e_base + k * B
    base = (k % 2) * B
    la = pltpu.async_copy(src_hbm.at[pl.ds(e0, B)], src_v, sem_a)
    lb = pltpu.async_copy(dst_hbm.at[pl.ds(e0, B)], dst_v, sem_b)
    la.wait()

    @plsc.parallel_loop(0, B // 16)
    def _o(i):
      src_v[pl.ds(i * 16, 16)] = src_v[pl.ds(i * 16, 16)] + offv

    g = pltpu.async_copy(
        feat2.at[src_v.at[pl.ds(0, B // 2)]],
        rows_v.at[pl.ds(0, B // 2)], sem_g)
    g2 = pltpu.async_copy(
        feat2.at[src_v.at[pl.ds(B // 2, B // 2)]],
        rows_v.at[pl.ds(B // 2, B // 2)], sem_h)
    _drain_eps()

    @plsc.parallel_loop(0, B, unroll=4)
    def _compute_w(r):
      for j in range(CG):
        eps_v[base + r, pl.ds(j * 16, 16)] = jnp.maximum(
            eps_v[base + r, pl.ds(j * 16, 16)] + 1.0, 0.0)

    lb.wait()
    sx = pltpu.async_copy(
        eps_v.at[pl.ds(base, B)], s_sh.at[dst_v], sem_x, add=True)
    sz = pltpu.async_copy(ones_v, deg_sh.at[dst_v], sem_z, add=True)
    _issue_eps(k + 1)
    g.wait()
    g2.wait()

    @plsc.parallel_loop(0, B, unroll=4)
    def _compute_m(r):
      for j in range(CG):
        msg_v[r, pl.ds(j * 16, 16)] = (
            eps_v[base + r, pl.ds(j * 16, 16)] * rows_v[r, pl.ds(j * 16, 16)])

    sy = pltpu.async_copy(msg_v, t_sh.at[dst_v], sem_y, add=True)
    sx.wait()
    sz.wait()
    sy.wait()
    return 0
  lax.fori_loop(0, NFULL, _chunk, 0)
  _drain_eps()

  e0 = e_base + NFULL * B
  pltpu.sync_copy(src_hbm.at[pl.ds(e0, TAIL)], srct_v)
  pltpu.sync_copy(dst_hbm.at[pl.ds(e0, TAIL)], dstt_v)
  pltpu.sync_copy(eps_r.at[pl.ds(e0, TAIL), pl.ds(ch0, 64)], epst_v)

  @plsc.parallel_loop(0, TAIL // 16)
  def _ot(i):
    srct_v[pl.ds(i * 16, 16)] = srct_v[pl.ds(i * 16, 16)] + offv

  pltpu.async_copy(feat2.at[srct_v], rowst_v, sem_g).wait()

  @plsc.parallel_loop(0, TAIL)
  def _compute_t(r):
    for j in range(CG):
      w = jnp.maximum(epst_v[r, pl.ds(j * 16, 16)] + 1.0, 0.0)
      epst_v[r, pl.ds(j * 16, 16)] = w
      msgt_v[r, pl.ds(j * 16, 16)] = w * rowst_v[r, pl.ds(j * 16, 16)]

  pltpu.sync_copy(epst_v, s_sh.at[dstt_v], add=True)
  pltpu.sync_copy(msgt_v, t_sh.at[dstt_v], add=True)
  pltpu.sync_copy(ones_v.at[pl.ds(0, TAIL)], deg_sh.at[dstt_v], add=True)

  plsc.subcore_barrier()

  pltpu.sync_copy(deg_sh.at[pl.ds(n0, NSLC)], deg_sm)

  def _norm(q, _):
    off = n0 + q * B
    pltpu.sync_copy(s_sh.at[pl.ds(off, B)], eps_v.at[pl.ds(0, B)])
    pltpu.sync_copy(t_sh.at[pl.ds(off, B)], rows_v)

    @plsc.parallel_loop(0, B)
    def _rows(r):
      d = jnp.full((16,), deg_sm[q * B + r], jnp.float32)
      for j in range(CG):
        s = eps_v[r, pl.ds(j * 16, 16)]
        nz = s != 0.0
        sc = jnp.where(nz, d / jnp.where(nz, s, 1.0), 0.0)
        msg_v[r, pl.ds(j * 16, 16)] = sc * rows_v[r, pl.ds(j * 16, 16)]

    pltpu.sync_copy(msg_v, out_r.at[pl.ds(off, B), pl.ds(ch0, 64)])
    return 0
  lax.fori_loop(0, NSLC // B, _norm, 0)


_sc_kernel = pl.kernel(
    _sc_body,
    out_type=jax.ShapeDtypeStruct((NPAD, 128), jnp.float32),
    mesh=plsc.VectorSubcoreMesh(core_axis_name="c", subcore_axis_name="s"),
    compiler_params=pltpu.CompilerParams(use_tc_tiling_on_sc=False),
    scratch_types=[
        pltpu.VMEM((2 * B, 64), jnp.float32),
        pltpu.VMEM((B, 64), jnp.float32),
        pltpu.VMEM((B, 64), jnp.float32),
        pltpu.VMEM((B,), jnp.int32),
        pltpu.VMEM((B,), jnp.int32),
        pltpu.VMEM((B,), jnp.float32),
        pltpu.VMEM((640,), jnp.float32),
        pltpu.VMEM((TAIL, 64), jnp.float32),
        pltpu.VMEM((TAIL, 64), jnp.float32),
        pltpu.VMEM((TAIL, 64), jnp.float32),
        pltpu.VMEM((TAIL,), jnp.int32),
        pltpu.VMEM((TAIL,), jnp.int32),
        pltpu.SMEM((NSLC,), jnp.float32),
        pltpu.VMEM_SHARED((NPAD, 64), jnp.float32),
        pltpu.VMEM_SHARED((NPAD, 64), jnp.float32),
        pltpu.VMEM_SHARED((NPAD,), jnp.float32),
        pltpu.SemaphoreType.DMA,
        pltpu.SemaphoreType.DMA,
        pltpu.SemaphoreType.DMA,
        pltpu.SemaphoreType.DMA,
        pltpu.SemaphoreType.DMA,
        pltpu.SemaphoreType.DMA,
        pltpu.SemaphoreType.DMA,
        pltpu.SemaphoreType.DMA,
    ],
)


def _mm_body(a_ref, w_ref, b_ref, o_ref):
  o_ref[...] = (
      jnp.dot(a_ref[...], w_ref[...], preferred_element_type=jnp.float32)
      + b_ref[0:1, :]
  )


def _matmul(agg_pad, W, b2):
  return pl.pallas_call(
      _mm_body,
      grid=(10,),
      in_specs=[
          pl.BlockSpec((1000, C), lambda i: (i, 0)),
          pl.BlockSpec((C, C), lambda i: (0, 0)),
          pl.BlockSpec((8, C), lambda i: (0, 0)),
      ],
      out_specs=pl.BlockSpec((1000, C), lambda i: (i, 0)),
      out_shape=jax.ShapeDtypeStruct((N, C), jnp.float32),
  )(agg_pad, W, b2)


def kernel(feat, edge_index, eps, W, b):
  feat_pad = jnp.concatenate(
      [feat, jnp.zeros((NPAD - N, C), feat.dtype)], axis=0)
  feat2 = jnp.concatenate(
      [feat_pad[:, :64], feat_pad[:, 64:]], axis=0)
  eps_r = eps
  agg = _sc_kernel(
      feat2, edge_index[0], edge_index[1], eps_r).reshape(NPAD, C)
  b2 = jnp.broadcast_to(b.reshape(1, C), (8, C))
  return _matmul(agg, W, b2)


# speedup vs baseline: 1.2228x; 1.0529x over previous
import jax
import jax.numpy as jnp
from jax import lax
from jax.experimental import pallas as pl
from jax.experimental.pallas import tpu as pltpu
from jax.experimental.pallas import tpu_sc as plsc

N = 10000
E = 320000
C = 128
NTILE = 16
NPAD = 10240
NSLC = NPAD // NTILE
EPT = E // NTILE
B = 128
NFULL = EPT // B
TAIL = EPT - NFULL * B
CG = 4


def _sc_body(feat2, src_hbm, dst_hbm, eps_r, out_r,
             eps_v, rows_v, msg_v, src_v, dst_v, dst_lo, dst_hi, ones_v, zer_v,
             epst_v, rowst_v, msgt_v, srct_v, dstt_v,
             deg_sm, s_sh, t_sh, deg_sh,
             sem_a, sem_b, sem_c, sem_g, sem_h, sem_x, sem_y, sem_y2, sem_z):
  core = lax.axis_index("c")
  sub = lax.axis_index("s")
  ch0 = core * 64
  n0 = sub * NSLC
  e_base = sub * EPT
  offv = jnp.full((16,), core * NPAD, jnp.int32)

  @plsc.parallel_loop(0, B)
  def _z(r):
    for j in range(CG):
      msg_v[r, pl.ds(j * 16, 16)] = jnp.zeros((16,), jnp.float32)

  @plsc.parallel_loop(0, 40)
  def _z1(i):
    zer_v[pl.ds(i * 16, 16)] = jnp.zeros((16,), jnp.float32)

  @plsc.parallel_loop(0, B // 16)
  def _z2(i):
    ones_v[pl.ds(i * 16, 16)] = jnp.ones((16,), jnp.float32)

  def _init(q, _):
    off = n0 + q * B
    pltpu.sync_copy(msg_v, s_sh.at[pl.ds(off, B)])
    pltpu.sync_copy(msg_v, t_sh.at[pl.ds(off, B)])
    return 0
  lax.fori_loop(0, NSLC // B, _init, 0)
  pltpu.sync_copy(zer_v.at[pl.ds(0, NSLC)], deg_sh.at[pl.ds(n0, NSLC)])

  plsc.subcore_barrier()

  def _issue_eps(kk):
    e0 = jnp.where(kk < NFULL, e_base + kk * B, e_base)
    base = (kk % 2) * B
    pltpu.async_copy(
        eps_r.at[pl.ds(e0, B), pl.ds(ch0, 64)], eps_v.at[pl.ds(base, B)],
        sem_c)

  def _drain_eps():
    pltpu.make_async_copy(
        eps_r.at[pl.ds(e_base, B), pl.ds(ch0, 64)],
        eps_v.at[pl.ds(0, B)], sem_c).wait()

  _issue_eps(0)

  def _chunk(k, _):
    e0 = e_base + k * B
    base = (k % 2) * B
    la = pltpu.async_copy(src_hbm.at[pl.ds(e0, B)], src_v, sem_a)
    lb = pltpu.async_copy(dst_hbm.at[pl.ds(e0, B)], dst_v, sem_b)
    la.wait()

    @plsc.parallel_loop(0, B // 16)
    def _o(i):
      src_v[pl.ds(i * 16, 16)] = src_v[pl.ds(i * 16, 16)] + offv

    g = pltpu.async_copy(
        feat2.at[src_v.at[pl.ds(0, B // 2)]],
        rows_v.at[pl.ds(0, B // 2)], sem_g)
    g2 = pltpu.async_copy(
        feat2.at[src_v.at[pl.ds(B // 2, B // 2)]],
        rows_v.at[pl.ds(B // 2, B // 2)], sem_h)
    _drain_eps()

    @plsc.parallel_loop(0, B, unroll=4)
    def _compute_w(r):
      for j in range(CG):
        eps_v[base + r, pl.ds(j * 16, 16)] = jnp.maximum(
            eps_v[base + r, pl.ds(j * 16, 16)] + 1.0, 0.0)

    lb.wait()

    @plsc.parallel_loop(0, 4)
    def _cp(i):
      dst_lo[pl.ds(i * 16, 16)] = dst_v[pl.ds(i * 16, 16)]
      dst_hi[pl.ds(i * 16, 16)] = dst_v[pl.ds(64 + i * 16, 16)]

    sx = pltpu.async_copy(
        eps_v.at[pl.ds(base, B)], s_sh.at[dst_v], sem_x, add=True)
    sz = pltpu.async_copy(ones_v, deg_sh.at[dst_v], sem_z, add=True)
    _issue_eps(k + 1)
    g.wait()
    g2.wait()

    @plsc.parallel_loop(0, B // 2, unroll=4)
    def _compute_m(r):
      for j in range(CG):
        msg_v[r, pl.ds(j * 16, 16)] = (
            eps_v[base + r, pl.ds(j * 16, 16)] * rows_v[r, pl.ds(j * 16, 16)])

    sy = pltpu.async_copy(
        msg_v.at[pl.ds(0, B // 2)], t_sh.at[dst_lo], sem_y, add=True)

    @plsc.parallel_loop(B // 2, B, unroll=4)
    def _compute_m2(r):
      for j in range(CG):
        msg_v[r, pl.ds(j * 16, 16)] = (
            eps_v[base + r, pl.ds(j * 16, 16)] * rows_v[r, pl.ds(j * 16, 16)])

    sy2 = pltpu.async_copy(
        msg_v.at[pl.ds(B // 2, B // 2)], t_sh.at[dst_hi], sem_y2, add=True)
    sx.wait()
    sz.wait()
    sy.wait()
    sy2.wait()
    return 0
  lax.fori_loop(0, NFULL, _chunk, 0)
  _drain_eps()

  e0 = e_base + NFULL * B
  pltpu.sync_copy(src_hbm.at[pl.ds(e0, TAIL)], srct_v)
  pltpu.sync_copy(dst_hbm.at[pl.ds(e0, TAIL)], dstt_v)
  pltpu.sync_copy(eps_r.at[pl.ds(e0, TAIL), pl.ds(ch0, 64)], epst_v)

  @plsc.parallel_loop(0, TAIL // 16)
  def _ot(i):
    srct_v[pl.ds(i * 16, 16)] = srct_v[pl.ds(i * 16, 16)] + offv

  pltpu.async_copy(feat2.at[srct_v], rowst_v, sem_g).wait()

  @plsc.parallel_loop(0, TAIL)
  def _compute_t(r):
    for j in range(CG):
      w = jnp.maximum(epst_v[r, pl.ds(j * 16, 16)] + 1.0, 0.0)
      epst_v[r, pl.ds(j * 16, 16)] = w
      msgt_v[r, pl.ds(j * 16, 16)] = w * rowst_v[r, pl.ds(j * 16, 16)]

  pltpu.sync_copy(epst_v, s_sh.at[dstt_v], add=True)
  pltpu.sync_copy(msgt_v, t_sh.at[dstt_v], add=True)
  pltpu.sync_copy(ones_v.at[pl.ds(0, TAIL)], deg_sh.at[dstt_v], add=True)

  plsc.subcore_barrier()

  pltpu.sync_copy(deg_sh.at[pl.ds(n0, NSLC)], deg_sm)

  def _norm(q, _):
    off = n0 + q * B
    pltpu.sync_copy(s_sh.at[pl.ds(off, B)], eps_v.at[pl.ds(0, B)])
    pltpu.sync_copy(t_sh.at[pl.ds(off, B)], rows_v)

    @plsc.parallel_loop(0, B)
    def _rows(r):
      d = jnp.full((16,), deg_sm[q * B + r], jnp.float32)
      for j in range(CG):
        s = eps_v[r, pl.ds(j * 16, 16)]
        nz = s != 0.0
        sc = jnp.where(nz, d / jnp.where(nz, s, 1.0), 0.0)
        msg_v[r, pl.ds(j * 16, 16)] = sc * rows_v[r, pl.ds(j * 16, 16)]

    pltpu.sync_copy(msg_v, out_r.at[pl.ds(off, B), pl.ds(ch0, 64)])
    return 0
  lax.fori_loop(0, NSLC // B, _norm, 0)


_sc_kernel = pl.kernel(
    _sc_body,
    out_type=jax.ShapeDtypeStruct((NPAD, 128), jnp.float32),
    mesh=plsc.VectorSubcoreMesh(core_axis_name="c", subcore_axis_name="s"),
    compiler_params=pltpu.CompilerParams(use_tc_tiling_on_sc=False),
    scratch_types=[
        pltpu.VMEM((2 * B, 64), jnp.float32),
        pltpu.VMEM((B, 64), jnp.float32),
        pltpu.VMEM((B, 64), jnp.float32),
        pltpu.VMEM((B,), jnp.int32),
        pltpu.VMEM((B,), jnp.int32),
        pltpu.VMEM((B // 2,), jnp.int32),
        pltpu.VMEM((B // 2,), jnp.int32),
        pltpu.VMEM((B,), jnp.float32),
        pltpu.VMEM((640,), jnp.float32),
        pltpu.VMEM((TAIL, 64), jnp.float32),
        pltpu.VMEM((TAIL, 64), jnp.float32),
        pltpu.VMEM((TAIL, 64), jnp.float32),
        pltpu.VMEM((TAIL,), jnp.int32),
        pltpu.VMEM((TAIL,), jnp.int32),
        pltpu.SMEM((NSLC,), jnp.float32),
        pltpu.VMEM_SHARED((NPAD, 64), jnp.float32),
        pltpu.VMEM_SHARED((NPAD, 64), jnp.float32),
        pltpu.VMEM_SHARED((NPAD,), jnp.float32),
        pltpu.SemaphoreType.DMA,
        pltpu.SemaphoreType.DMA,
        pltpu.SemaphoreType.DMA,
        pltpu.SemaphoreType.DMA,
        pltpu.SemaphoreType.DMA,
        pltpu.SemaphoreType.DMA,
        pltpu.SemaphoreType.DMA,
        pltpu.SemaphoreType.DMA,
        pltpu.SemaphoreType.DMA,
    ],
)


def _mm_body(a_ref, w_ref, b_ref, o_ref):
  o_ref[...] = (
      jnp.dot(a_ref[...], w_ref[...], preferred_element_type=jnp.float32)
      + b_ref[0:1, :]
  )


def _matmul(agg_pad, W, b2):
  return pl.pallas_call(
      _mm_body,
      grid=(10,),
      in_specs=[
          pl.BlockSpec((1000, C), lambda i: (i, 0)),
          pl.BlockSpec((C, C), lambda i: (0, 0)),
          pl.BlockSpec((8, C), lambda i: (0, 0)),
      ],
      out_specs=pl.BlockSpec((1000, C), lambda i: (i, 0)),
      out_shape=jax.ShapeDtypeStruct((N, C), jnp.float32),
  )(agg_pad, W, b2)


def kernel(feat, edge_index, eps, W, b):
  feat_pad = jnp.concatenate(
      [feat, jnp.zeros((NPAD - N, C), feat.dtype)], axis=0)
  feat2 = jnp.concatenate(
      [feat_pad[:, :64], feat_pad[:, 64:]], axis=0)
  eps_r = eps
  agg = _sc_kernel(
      feat2, edge_index[0], edge_index[1], eps_r).reshape(NPAD, C)
  b2 = jnp.broadcast_to(b.reshape(1, C), (8, C))
  return _matmul(agg, W, b2)
